# Initial kernel scaffold; baseline (speedup 1.0000x reference)
import jax
import jax.numpy as jnp
from jax.experimental import pallas as pl


def kernel(x, nn_idx, etype, filters1, filters2, bias, bn_gamma, bn_beta):
    raise NotImplementedError("write your pallas kernel here")


# speedup vs baseline: 7.6671x; 7.6671x over previous
import functools

import jax
import jax.numpy as jnp
from jax import lax
from jax.experimental import pallas as pl
from jax.experimental.pallas import tpu as pltpu
from jax.experimental.pallas import tpu_sc as plsc

_NIN = 256
_NOU = 128
_ET = 4
_N = 10000
_K = 16
_D = _NOU * _ET

_NB_MM = 2000


def _mm_body(x_ref, wn_ref, we_ref, ef_ref, s_ref):
    xb = x_ref[...]
    dn = (((1,), (0,)), ((), ()))
    nf = lax.dot_general(xb, wn_ref[...], dn, preferred_element_type=jnp.float32)
    ef = lax.dot_general(xb, we_ref[...], dn, preferred_element_type=jnp.float32)
    ef_ref[...] = ef
    s_ref[...] = nf + ef


def _matmul(x, wn, we):
    return pl.pallas_call(
        _mm_body,
        grid=(_N // _NB_MM,),
        in_specs=[
            pl.BlockSpec((_NB_MM, _NIN), lambda i: (i, 0)),
            pl.BlockSpec((_NIN, _D), lambda i: (0, 0)),
            pl.BlockSpec((_NIN, _D), lambda i: (0, 0)),
        ],
        out_specs=[
            pl.BlockSpec((_NB_MM, _D), lambda i: (i, 0)),
            pl.BlockSpec((_NB_MM, _D), lambda i: (i, 0)),
        ],
        out_shape=[
            jax.ShapeDtypeStruct((_N, _D), jnp.float32),
            jax.ShapeDtypeStruct((_N, _D), jnp.float32),
        ],
    )(x, wn, we)


_NC = 2
_NS = 16
_NW = _NC * _NS
_CH = 120
_NCHUNK = 42
_BPW = _CH * _NCHUNK
_EP = _BPW * _NW


def _make_gather():
    mesh = plsc.VectorSubcoreMesh(core_axis_name="c", subcore_axis_name="s")

    @functools.partial(
        pl.kernel,
        mesh=mesh,
        out_type=jax.ShapeDtypeStruct((_EP, _D), jnp.float32),
        scratch_types=[
            pltpu.VMEM((2, _CH), jnp.int32),
            pltpu.VMEM((2, _CH, _D), jnp.float32),
            pltpu.SemaphoreType.DMA,
            pltpu.SemaphoreType.DMA((2,)),
        ],
    )
    def gather_k(table_hbm, idx_hbm, out_hbm, idx_v, rows_v, gsem, wsem):
        wid = lax.axis_index("s") * _NC + lax.axis_index("c")
        base = wid * _BPW

        def chunk(g, slot):
            off = base + g * _CH

            @pl.when(g >= 2)
            def _wait_prev_write():
                pltpu.make_async_copy(
                    rows_v.at[slot], out_hbm.at[pl.ds(0, _CH)], wsem.at[slot]
                ).wait()

            pltpu.sync_copy(idx_hbm.at[pl.ds(off, _CH)], idx_v.at[slot])
            pltpu.async_copy(
                table_hbm.at[idx_v.at[slot]], rows_v.at[slot], gsem
            ).wait()
            pltpu.async_copy(rows_v.at[slot], out_hbm.at[pl.ds(off, _CH)], wsem.at[slot])

        def outer(g2, carry):
            chunk(g2 * 2, 0)
            chunk(g2 * 2 + 1, 1)
            return carry

        lax.fori_loop(0, _NCHUNK // 2, outer, 0)
        for b in range(2):
            pltpu.make_async_copy(
                rows_v.at[b], out_hbm.at[pl.ds(0, _CH)], wsem.at[b]
            ).wait()

    return gather_k


_gather_cache = []


def _gather(table, idx):
    if not _gather_cache:
        _gather_cache.append(_make_gather())
    return _gather_cache[0](table, idx)


_NB_E = 200
_RB = _NB_E * _K


def _edge_body(s_ref, pts_ref, et_ref, bias_ref, out_ref):
    s = s_ref[...]
    pts = pts_ref[...]
    et = et_ref[...]
    acc = jnp.zeros((_RB, _NOU), jnp.float32)
    for e in range(_ET):
        se = s[:, e * _NOU:(e + 1) * _NOU]
        s_bc = jnp.broadcast_to(se[:, None, :], (_NB_E, _K, _NOU)).reshape(_RB, _NOU)
        pe = pts[:, e * _NOU:(e + 1) * _NOU]
        w = et[:, e:e + 1]
        acc = acc + w * (s_bc - pe)
    acc = acc + bias_ref[...]
    out_ref[...] = jnp.max(acc.reshape(_NB_E, _K, _NOU), axis=1)


def _edge(s, pts, et_t, bias2d):
    return pl.pallas_call(
        _edge_body,
        grid=(_N // _NB_E,),
        in_specs=[
            pl.BlockSpec((_NB_E, _D), lambda i: (i, 0)),
            pl.BlockSpec((_RB, _D), lambda i: (i, 0)),
            pl.BlockSpec((_RB, _ET), lambda i: (i, 0)),
            pl.BlockSpec((1, _NOU), lambda i: (0, 0)),
        ],
        out_specs=pl.BlockSpec((_NB_E, _NOU), lambda i: (i, 0)),
        out_shape=jax.ShapeDtypeStruct((_N, _NOU), jnp.float32),
    )(s, pts, et_t, bias2d)


def _bn_body(o_ref, g_ref, b_ref, out_ref):
    o = o_ref[...]
    mean = jnp.mean(o, axis=0, keepdims=True)
    ctr = o - mean
    var = jnp.mean(ctr * ctr, axis=0, keepdims=True)
    y = ctr * lax.rsqrt(var + 1e-5) * g_ref[...] + b_ref[...]
    out_ref[...] = jnp.maximum(y, 0.0).T


def _bn(o, g, b):
    return pl.pallas_call(
        _bn_body,
        out_shape=jax.ShapeDtypeStruct((_NOU, _N), jnp.float32),
    )(o, g, b)


def kernel(x, nn_idx, etype, filters1, filters2, bias, bn_gamma, bn_beta):
    wn = jnp.transpose(filters1, (0, 2, 1)).reshape(_NIN, _D)
    we = jnp.transpose(filters2, (0, 2, 1)).reshape(_NIN, _D)
    efeat, s = _matmul(jnp.transpose(x.reshape(_NIN, _N)), wn, we)

    idx = nn_idx.reshape(-1).astype(jnp.int32)
    idx = jnp.concatenate([idx, jnp.zeros((_EP - _N * _K,), jnp.int32)])
    pts = _gather(efeat, idx)

    et_t = jnp.transpose(etype.reshape(_ET, _N * _K))
    out_pre = _edge(s, pts, et_t, bias.reshape(1, _NOU))
    out = _bn(out_pre, bn_gamma.reshape(1, _NOU), bn_beta.reshape(1, _NOU))
    return out.reshape(1, _NOU, _N, 1)


# speedup vs baseline: 8.7737x; 1.1443x over previous
import functools

import jax
import jax.numpy as jnp
from jax import lax
from jax.experimental import pallas as pl
from jax.experimental.pallas import tpu as pltpu
from jax.experimental.pallas import tpu_sc as plsc

_NIN = 256
_NOU = 128
_ET = 4
_N = 10000
_K = 16
_D = _NOU * _ET

_NB_MM = 2000


def _mm_body(x_ref, wn_ref, we_ref, ef_ref, s_ref):
    xb = x_ref[...]
    dn = (((1,), (0,)), ((), ()))
    nf = lax.dot_general(xb, wn_ref[...], dn, preferred_element_type=jnp.float32)
    ef = lax.dot_general(xb, we_ref[...], dn, preferred_element_type=jnp.float32)
    ef_ref[...] = ef
    s_ref[...] = nf + ef


def _matmul(x, wn, we):
    return pl.pallas_call(
        _mm_body,
        grid=(_N // _NB_MM,),
        in_specs=[
            pl.BlockSpec((_NB_MM, _NIN), lambda i: (i, 0)),
            pl.BlockSpec((_NIN, _D), lambda i: (0, 0)),
            pl.BlockSpec((_NIN, _D), lambda i: (0, 0)),
        ],
        out_specs=[
            pl.BlockSpec((_NB_MM, _D), lambda i: (i, 0)),
            pl.BlockSpec((_NB_MM, _D), lambda i: (i, 0)),
        ],
        out_shape=[
            jax.ShapeDtypeStruct((_N, _D), jnp.float32),
            jax.ShapeDtypeStruct((_N, _D), jnp.float32),
        ],
    )(x, wn, we)


_NC = 2
_NS = 16
_NW = _NC * _NS
_CH = 80
_NCHUNK = 63
_BPW = _CH * _NCHUNK
_EP = _BPW * _NW
_NPAD = _EP // _K


def _make_gather():
    mesh = plsc.VectorSubcoreMesh(core_axis_name="c", subcore_axis_name="s")

    @functools.partial(
        pl.kernel,
        mesh=mesh,
        out_type=jax.ShapeDtypeStruct((_EP, _D), jnp.float32),
        scratch_types=[
            pltpu.VMEM((_BPW,), jnp.int32),
            pltpu.VMEM((3, _CH, _D), jnp.float32),
            pltpu.SemaphoreType.DMA((3,)),
            pltpu.SemaphoreType.DMA((3,)),
        ],
    )
    def gather_k(table_hbm, idx_hbm, out_hbm, idx_v, rows_v, gsem, wsem):
        wid = lax.axis_index("s") * _NC + lax.axis_index("c")
        base = wid * _BPW
        pltpu.sync_copy(idx_hbm.at[pl.ds(base, _BPW)], idx_v)

        def fire_gather(g, slot):
            pltpu.async_copy(
                table_hbm.at[idx_v.at[pl.ds(g * _CH, _CH)]],
                rows_v.at[slot],
                gsem.at[slot],
            )

        def wait_gather(g, slot):
            pltpu.make_async_copy(
                table_hbm.at[idx_v.at[pl.ds(g * _CH, _CH)]],
                rows_v.at[slot],
                gsem.at[slot],
            ).wait()

        def fire_write(g, slot):
            pltpu.async_copy(
                rows_v.at[slot], out_hbm.at[pl.ds(base + g * _CH, _CH)], wsem.at[slot]
            )

        def wait_write(slot):
            pltpu.make_async_copy(
                rows_v.at[slot], out_hbm.at[pl.ds(0, _CH)], wsem.at[slot]
            ).wait()

        fire_gather(0, 0)
        fire_gather(1, 1)

        def outer(g3, carry):
            for b in range(3):
                g = g3 * 3 + b
                wait_gather(g, b)
                fire_write(g, b)
                nxt = (b + 2) % 3
                if b == 0:
                    @pl.when(g3 >= 1)
                    def _():
                        wait_write(nxt)
                    fire_gather(g + 2, nxt)
                else:
                    @pl.when(g3 <= 19)
                    def _():
                        wait_write(nxt)
                        fire_gather(g + 2, nxt)
            return carry

        lax.fori_loop(0, _NCHUNK // 3, outer, 0)
        for b in range(3):
            wait_write(b)

    return gather_k


_gather_cache = []


def _gather(table, idx):
    if not _gather_cache:
        _gather_cache.append(_make_gather())
    return _gather_cache[0](table, idx)


_NB_E = 200


def _edge_body(s_ref, pts_ref, et_ref, bias_ref, out_ref):
    s = s_ref[...]
    m = None
    for k in range(_K):
        pk = pts_ref[k]
        ek = et_ref[k]
        t = jnp.zeros((_NB_E, _NOU), jnp.float32)
        for e in range(_ET):
            w = ek[:, e:e + 1]
            t = t + w * (s[:, e * _NOU:(e + 1) * _NOU] - pk[:, e * _NOU:(e + 1) * _NOU])
        m = t if m is None else jnp.maximum(m, t)
    out_ref[...] = m + bias_ref[...]


def _edge(s, pts_km, et_km, bias2d):
    return pl.pallas_call(
        _edge_body,
        grid=(_N // _NB_E,),
        in_specs=[
            pl.BlockSpec((_NB_E, _D), lambda i: (i, 0)),
            pl.BlockSpec((_K, _NB_E, _D), lambda i: (0, i, 0)),
            pl.BlockSpec((_K, _NB_E, _ET), lambda i: (0, i, 0)),
            pl.BlockSpec((1, _NOU), lambda i: (0, 0)),
        ],
        out_specs=pl.BlockSpec((_NB_E, _NOU), lambda i: (i, 0)),
        out_shape=jax.ShapeDtypeStruct((_N, _NOU), jnp.float32),
    )(s, pts_km, et_km, bias2d)


def _bn_body(o_ref, g_ref, b_ref, out_ref):
    o = o_ref[...]
    mean = jnp.mean(o, axis=0, keepdims=True)
    ctr = o - mean
    var = jnp.mean(ctr * ctr, axis=0, keepdims=True)
    y = ctr * lax.rsqrt(var + 1e-5) * g_ref[...] + b_ref[...]
    out_ref[...] = jnp.maximum(y, 0.0).T


def _bn(o, g, b):
    return pl.pallas_call(
        _bn_body,
        out_shape=jax.ShapeDtypeStruct((_NOU, _N), jnp.float32),
    )(o, g, b)


def kernel(x, nn_idx, etype, filters1, filters2, bias, bn_gamma, bn_beta):
    wn = jnp.transpose(filters1, (0, 2, 1)).reshape(_NIN, _D)
    we = jnp.transpose(filters2, (0, 2, 1)).reshape(_NIN, _D)
    efeat, s = _matmul(jnp.transpose(x.reshape(_NIN, _N)), wn, we)

    pad = (jnp.arange(_K * (_NPAD - _N), dtype=jnp.int32) * 7) % _N
    idx_km = jnp.concatenate(
        [
            jnp.transpose(nn_idx.reshape(_N, _K)).astype(jnp.int32),
            pad.reshape(_K, _NPAD - _N),
        ],
        axis=1,
    ).reshape(-1)
    pts_km = _gather(efeat, idx_km).reshape(_K, _NPAD, _D)

    et_km = jnp.transpose(etype.reshape(_ET, _N, _K), (2, 1, 0))
    out_pre = _edge(s, pts_km, et_km, bias.reshape(1, _NOU))
    out = _bn(out_pre, bn_gamma.reshape(1, _NOU), bn_beta.reshape(1, _NOU))
    return out.reshape(1, _NOU, _N, 1)
